# baseline (speedup 1.0000x reference)
import functools

import jax
import jax.numpy as jnp
from jax import lax
from jax.experimental import pallas as pl
from jax.experimental.pallas import tpu as pltpu
from jax.experimental.pallas import tpu_sc as plsc

NC = 2
NS = 16
LN = 16
CH = 64
BLK = 8
RING = 4
LW = 128


def _pad_geometry(N, E):
    NW = NC * NS
    Np = -(-N // (NS * 8)) * (NS * 8)
    n_ch = -(-(-(-E // NW) // CH) // (2 * BLK)) * 2 * BLK
    return NW, Np, n_ch


def _make_sc_agg(N, E, D, with_hist):
    NW, Np, n_ch = _pad_geometry(N, E)
    n_blk = n_ch // BLK
    pairs = n_blk // 2
    rows_per_tile = Np // NS
    NR = Np // LW
    NRp = -(-(NR + 1) // 8) * 8
    e_pw = n_ch * CH
    IB = BLK * CH
    FW = E // e_pw

    mesh = plsc.VectorSubcoreMesh(core_axis_name="c", subcore_axis_name="s")

    out_type = [jax.ShapeDtypeStruct((NC, Np, D), jnp.float32)]
    scratch = (
        [pltpu.VMEM((IB,), jnp.int32)] * 4
        + [pltpu.VMEM((CH, D), jnp.float32)] * RING
        + [pltpu.VMEM_SHARED((Np, D), jnp.float32)]
        + [pltpu.SemaphoreType.DMA] * (2 * RING + 2)
    )
    if with_hist:
        out_type.append(jax.ShapeDtypeStruct((NW, NRp, LW), jnp.float32))
        scratch.append(pltpu.VMEM((NRp, LW), jnp.float32))

    @functools.partial(
        pl.kernel,
        mesh=mesh,
        out_type=tuple(out_type),
        compiler_params=pltpu.CompilerParams(needs_layout_passes=False),
        scratch_types=scratch,
    )
    def agg(table_hbm, edge_hbm, srcp_hbm, dstp_hbm, *rest):
        rest = list(rest)
        out_hbm = rest.pop(0)
        hist_hbm = rest.pop(0) if with_hist else None
        hist = rest.pop() if with_hist else None
        sib0, sib1, dib0, dib1 = rest[:4]
        rows = rest[4:4 + RING]
        accum_sh = rest[4 + RING]
        sems = rest[5 + RING:]
        gsem = sems[:RING]
        ssem = sems[RING:2 * RING]
        is0, is1 = sems[2 * RING:2 * RING + 2]
        c = lax.axis_index("c")
        s = lax.axis_index("s")
        wid = s * NC + c
        row0 = s * rows_per_tile
        r0 = rows[0]
        zero16 = jnp.zeros((LN,), jnp.float32)
        ones16 = jnp.ones((LN,), jnp.float32)

        def zero_row(r, carry):
            for q in range(D // LN):
                r0[r, pl.ds(q * LN, LN)] = zero16
            return carry

        lax.fori_loop(0, CH, zero_row, 0)
        n_full = rows_per_tile // CH
        for i in range(n_full):
            pltpu.sync_copy(r0, accum_sh.at[pl.ds(row0 + i * CH, CH)])
        tail = rows_per_tile - n_full * CH
        if tail:
            pltpu.sync_copy(r0.at[pl.ds(0, tail)],
                            accum_sh.at[pl.ds(row0 + n_full * CH, tail)])
        if with_hist:
            def zero_hrow(r, carry):
                for q in range(LW // LN):
                    hist[r, pl.ds(q * LN, LN)] = zero16
                return carry

            lax.fori_loop(0, NRp, zero_hrow, 0)
        plsc.subcore_barrier()

        def g_start(ib, k, sl):
            pltpu.async_copy(table_hbm.at[ib.at[pl.ds(k * CH, CH)]],
                             rows[sl], gsem[sl])

        def g_wait(sl):
            pltpu.make_async_copy(table_hbm.at[pl.ds(0, CH)], rows[sl],
                                  gsem[sl]).wait()

        def s_start(ib, k, sl):
            pltpu.async_copy(rows[sl], accum_sh.at[ib.at[pl.ds(k * CH, CH)]],
                             ssem[sl], add=True)

        def s_wait(sl):
            pltpu.make_async_copy(table_hbm.at[pl.ds(0, CH)], rows[sl],
                                  ssem[sl]).wait()

        def i_start(b, sib, dib, sem):
            @pl.when(wid < FW)
            def _():
                off = wid * e_pw + b * IB
                pltpu.async_copy(edge_hbm.at[0, pl.ds(off, IB)], sib, sem)
                pltpu.async_copy(edge_hbm.at[1, pl.ds(off, IB)], dib, sem)

            @pl.when(wid >= FW)
            def _():
                off = (wid - FW) * e_pw + b * IB
                pltpu.async_copy(srcp_hbm.at[pl.ds(off, IB)], sib, sem)
                pltpu.async_copy(dstp_hbm.at[pl.ds(off, IB)], dib, sem)

        def i_wait(sib, dib, sem):
            pltpu.make_async_copy(srcp_hbm.at[pl.ds(0, IB)], sib, sem).wait()
            pltpu.make_async_copy(srcp_hbm.at[pl.ds(0, IB)], dib, sem).wait()

        def histo(dib):
            for k in range(IB // LN):
                idx16 = dib[pl.ds(k * LN, LN)]
                hi = lax.shift_right_logical(idx16, 7)
                lo = lax.bitwise_and(idx16, 127)
                plsc.addupdate_scatter(hist, [hi, lo], ones16)

        def block(sib, dib, waits_in, drain_out):
            for k in range(RING):
                if waits_in:
                    s_wait(k)
                g_start(sib, k, k)
            if with_hist:
                histo(dib)
            for k in range(BLK):
                sl = k % RING
                g_wait(sl)
                s_start(dib, k, sl)
                if k + RING < BLK:
                    s_wait(sl)
                    g_start(sib, k + RING, sl)
            if drain_out:
                for k in range(RING):
                    s_wait(k)

        i_start(0, sib0, dib0, is0)
        i_start(1, sib1, dib1, is1)

        def pair_body(p, carry):
            i_wait(sib0, dib0, is0)
            block(sib0, dib0, waits_in=False, drain_out=False)
            i_wait(sib1, dib1, is1)
            block(sib1, dib1, waits_in=True, drain_out=True)

            @pl.when(p < pairs - 1)
            def _():
                i_start(2 * p + 2, sib0, dib0, is0)
                i_start(2 * p + 3, sib1, dib1, is1)

            return carry

        lax.fori_loop(0, pairs, pair_body, 0)

        plsc.subcore_barrier()
        pltpu.sync_copy(accum_sh.at[pl.ds(row0, rows_per_tile)],
                        out_hbm.at[c, pl.ds(row0, rows_per_tile)])
        if with_hist:
            pltpu.sync_copy(hist, hist_hbm.at[wid])

    return agg


def _dot_t(a, w):
    return lax.dot_general(a, w, (((1,), (1,)), ((), ())),
                           preferred_element_type=jnp.float32)


def _make_tc_layer(N, D, H, relu, R=1024):
    def body(sums_ref, cnt_ref, x_ref, wl_ref, bl_ref, wr_ref, o_ref):
        sm = sums_ref[0] + sums_ref[1]
        cnt = jnp.maximum(cnt_ref[...], 1.0)
        agg = sm / cnt
        out = (_dot_t(agg, wl_ref[...]) + bl_ref[...]
               + _dot_t(x_ref[...], wr_ref[...]))
        if relu:
            out = jnp.maximum(out, 0.0)
        o_ref[...] = out

    return pl.pallas_call(
        body,
        grid=(-(-N // R),),
        in_specs=[
            pl.BlockSpec((NC, R, D), lambda i: (0, i, 0)),
            pl.BlockSpec((R, 1), lambda i: (i, 0)),
            pl.BlockSpec((R, D), lambda i: (i, 0)),
            pl.BlockSpec((H, D), lambda i: (0, 0)),
            pl.BlockSpec((1, H), lambda i: (0, 0)),
            pl.BlockSpec((H, D), lambda i: (0, 0)),
        ],
        out_specs=pl.BlockSpec((R, H), lambda i: (i, 0)),
        out_shape=jax.ShapeDtypeStruct((N, H), jnp.float32),
    )


@jax.jit
def kernel(x, edge_index, W_l1, b_l1, W_r1, W_l2, b_l2, W_r2):
    N, D = x.shape
    E = edge_index.shape[1]
    H = W_l1.shape[0]
    NW, Np, n_ch = _pad_geometry(N, E)
    e_pw = n_ch * CH
    FW = E // e_pw
    n_pad = NW * e_pw - E
    pad_src = jnp.arange(n_pad, dtype=jnp.int32) % N
    pad_dst = N + jnp.arange(n_pad, dtype=jnp.int32) % (Np - N)
    srcp = jnp.concatenate([edge_index[0, FW * e_pw:], pad_src])
    dstp = jnp.concatenate([edge_index[1, FW * e_pw:], pad_dst])

    sc_agg1 = _make_sc_agg(N, E, D, with_hist=True)
    sc_agg2 = _make_sc_agg(N, E, H, with_hist=False)
    tc1 = _make_tc_layer(N, D, H, relu=True)
    tc2 = _make_tc_layer(N, H, H, relu=False)

    sums1, hist = sc_agg1(x, edge_index, srcp, dstp)
    cnt = jnp.sum(hist, axis=0).reshape(-1, 1)
    h1 = tc1(sums1, cnt, x, W_l1, b_l1[None, :], W_r1)
    (sums2,) = sc_agg2(h1, edge_index, srcp, dstp)
    out = tc2(sums2, cnt, h1, W_l2, b_l2[None, :], W_r2)
    return out


# speedup vs baseline: 1.0403x; 1.0403x over previous
import functools

import jax
import jax.numpy as jnp
from jax import lax
from jax.experimental import pallas as pl
from jax.experimental.pallas import tpu as pltpu
from jax.experimental.pallas import tpu_sc as plsc

NC = 2
NS = 16
LN = 16
CH = 128
BLK = 8
RING = 2


def _pad_geometry(N, E):
    NW = NC * NS
    Np = -(-N // (NS * 8)) * (NS * 8)
    n_ch = -(-(-(-E // NW) // CH) // (2 * BLK)) * 2 * BLK
    return NW, Np, n_ch


def _make_sc_agg(N, E, D, with_hist):
    NW, Np, n_ch = _pad_geometry(N, E)
    n_blk = n_ch // BLK
    pairs = n_blk // 2
    rows_per_tile = Np // NS
    NR = Np // CH
    NRp = -(-(NR + 1) // 8) * 8
    e_pw = n_ch * CH
    IB = BLK * CH
    FW = E // e_pw

    mesh = plsc.VectorSubcoreMesh(core_axis_name="c", subcore_axis_name="s")

    out_type = [jax.ShapeDtypeStruct((NC, Np, D), jnp.float32)]
    scratch = [
        pltpu.VMEM((IB,), jnp.int32),
        pltpu.VMEM((IB,), jnp.int32),
        pltpu.VMEM((IB,), jnp.int32),
        pltpu.VMEM((IB,), jnp.int32),
        pltpu.VMEM((CH, D), jnp.float32),
        pltpu.VMEM((CH, D), jnp.float32),
        pltpu.VMEM_SHARED((Np, D), jnp.float32),
        pltpu.SemaphoreType.DMA,
        pltpu.SemaphoreType.DMA,
        pltpu.SemaphoreType.DMA,
        pltpu.SemaphoreType.DMA,
        pltpu.SemaphoreType.DMA,
        pltpu.SemaphoreType.DMA,
    ]
    if with_hist:
        out_type.append(jax.ShapeDtypeStruct((NW, NRp, CH), jnp.float32))
        scratch.append(pltpu.VMEM((NRp, CH), jnp.float32))

    @functools.partial(
        pl.kernel,
        mesh=mesh,
        out_type=tuple(out_type),
        compiler_params=pltpu.CompilerParams(needs_layout_passes=False),
        scratch_types=scratch,
    )
    def agg(table_hbm, edge_hbm, srcp_hbm, dstp_hbm, *rest):
        if with_hist:
            (out_hbm, hist_hbm, sib0, sib1, dib0, dib1, r0, r1, accum_sh,
             gs0, gs1, ss0, ss1, is0, is1, hist) = rest
        else:
            (out_hbm, sib0, sib1, dib0, dib1, r0, r1, accum_sh,
             gs0, gs1, ss0, ss1, is0, is1) = rest
            hist = None
        c = lax.axis_index("c")
        s = lax.axis_index("s")
        wid = s * NC + c
        row0 = s * rows_per_tile
        rows = [r0, r1]
        gsem = [gs0, gs1]
        ssem = [ss0, ss1]
        zero16 = jnp.zeros((LN,), jnp.float32)
        ones16 = jnp.ones((LN,), jnp.float32)

        def zero_row(r, carry):
            for q in range(D // LN):
                r0[r, pl.ds(q * LN, LN)] = zero16
            return carry

        lax.fori_loop(0, CH, zero_row, 0)
        n_full = rows_per_tile // CH
        for i in range(n_full):
            pltpu.sync_copy(r0, accum_sh.at[pl.ds(row0 + i * CH, CH)])
        tail = rows_per_tile - n_full * CH
        if tail:
            pltpu.sync_copy(r0.at[pl.ds(0, tail)],
                            accum_sh.at[pl.ds(row0 + n_full * CH, tail)])
        if with_hist:
            def zero_hrow(r, carry):
                for q in range(CH // LN):
                    hist[r, pl.ds(q * LN, LN)] = zero16
                return carry

            lax.fori_loop(0, NRp, zero_hrow, 0)
        plsc.subcore_barrier()

        def g_start(ib, k, sl):
            pltpu.async_copy(table_hbm.at[ib.at[pl.ds(k * CH, CH)]],
                             rows[sl], gsem[sl])

        def g_wait(sl):
            pltpu.make_async_copy(table_hbm.at[pl.ds(0, CH)], rows[sl],
                                  gsem[sl]).wait()

        def s_start(ib, k, sl):
            pltpu.async_copy(rows[sl], accum_sh.at[ib.at[pl.ds(k * CH, CH)]],
                             ssem[sl], add=True)

        def s_wait(sl):
            pltpu.make_async_copy(table_hbm.at[pl.ds(0, CH)], rows[sl],
                                  ssem[sl]).wait()

        def i_start(b, sib, dib, sem):
            @pl.when(wid < FW)
            def _():
                off = wid * e_pw + b * IB
                pltpu.async_copy(edge_hbm.at[0, pl.ds(off, IB)], sib, sem)
                pltpu.async_copy(edge_hbm.at[1, pl.ds(off, IB)], dib, sem)

            @pl.when(wid >= FW)
            def _():
                off = (wid - FW) * e_pw + b * IB
                pltpu.async_copy(srcp_hbm.at[pl.ds(off, IB)], sib, sem)
                pltpu.async_copy(dstp_hbm.at[pl.ds(off, IB)], dib, sem)

        def i_wait(sib, dib, sem):
            pltpu.make_async_copy(srcp_hbm.at[pl.ds(0, IB)], sib, sem).wait()
            pltpu.make_async_copy(srcp_hbm.at[pl.ds(0, IB)], dib, sem).wait()

        def histo(dib):
            for k in range(IB // LN):
                idx16 = dib[pl.ds(k * LN, LN)]
                hi = lax.shift_right_logical(idx16, 7)
                lo = lax.bitwise_and(idx16, 127)
                plsc.addupdate_scatter(hist, [hi, lo], ones16)

        def block(sib, dib, waits_in, drain_out):
            for k in range(RING):
                if waits_in:
                    s_wait(k)
                g_start(sib, k, k)
            if with_hist:
                histo(dib)
            for k in range(BLK):
                sl = k % RING
                g_wait(sl)
                s_start(dib, k, sl)
                if k + RING < BLK:
                    s_wait(sl)
                    g_start(sib, k + RING, sl)
            if drain_out:
                for k in range(RING):
                    s_wait(k)

        i_start(0, sib0, dib0, is0)
        i_start(1, sib1, dib1, is1)

        def pair_body(p, carry):
            i_wait(sib0, dib0, is0)
            block(sib0, dib0, waits_in=False, drain_out=False)
            i_wait(sib1, dib1, is1)
            block(sib1, dib1, waits_in=True, drain_out=True)

            @pl.when(p < pairs - 1)
            def _():
                i_start(2 * p + 2, sib0, dib0, is0)
                i_start(2 * p + 3, sib1, dib1, is1)

            return carry

        lax.fori_loop(0, pairs, pair_body, 0)

        plsc.subcore_barrier()
        pltpu.sync_copy(accum_sh.at[pl.ds(row0, rows_per_tile)],
                        out_hbm.at[c, pl.ds(row0, rows_per_tile)])
        if with_hist:
            pltpu.sync_copy(hist, hist_hbm.at[wid])

    return agg


def _dot_t(a, w):
    return lax.dot_general(a, w, (((1,), (1,)), ((), ())),
                           preferred_element_type=jnp.float32)


def _make_tc_layer(N, D, H, relu, R=1024):
    def body(sums_ref, cnt_ref, x_ref, wl_ref, bl_ref, wr_ref, o_ref):
        sm = sums_ref[0] + sums_ref[1]
        cnt = jnp.maximum(cnt_ref[...], 1.0)
        agg = sm / cnt
        out = (_dot_t(agg, wl_ref[...]) + bl_ref[...]
               + _dot_t(x_ref[...], wr_ref[...]))
        if relu:
            out = jnp.maximum(out, 0.0)
        o_ref[...] = out

    return pl.pallas_call(
        body,
        grid=(-(-N // R),),
        in_specs=[
            pl.BlockSpec((NC, R, D), lambda i: (0, i, 0)),
            pl.BlockSpec((R, 1), lambda i: (i, 0)),
            pl.BlockSpec((R, D), lambda i: (i, 0)),
            pl.BlockSpec((H, D), lambda i: (0, 0)),
            pl.BlockSpec((1, H), lambda i: (0, 0)),
            pl.BlockSpec((H, D), lambda i: (0, 0)),
        ],
        out_specs=pl.BlockSpec((R, H), lambda i: (i, 0)),
        out_shape=jax.ShapeDtypeStruct((N, H), jnp.float32),
    )


@jax.jit
def kernel(x, edge_index, W_l1, b_l1, W_r1, W_l2, b_l2, W_r2):
    N, D = x.shape
    E = edge_index.shape[1]
    H = W_l1.shape[0]
    NW, Np, n_ch = _pad_geometry(N, E)
    e_pw = n_ch * CH
    FW = E // e_pw
    n_pad = NW * e_pw - E
    pad_src = jnp.arange(n_pad, dtype=jnp.int32) % N
    pad_dst = N + jnp.arange(n_pad, dtype=jnp.int32) % (Np - N)
    srcp = jnp.concatenate([edge_index[0, FW * e_pw:], pad_src])
    dstp = jnp.concatenate([edge_index[1, FW * e_pw:], pad_dst])

    sc_agg1 = _make_sc_agg(N, E, D, with_hist=True)
    sc_agg2 = _make_sc_agg(N, E, H, with_hist=False)
    tc1 = _make_tc_layer(N, D, H, relu=True)
    tc2 = _make_tc_layer(N, H, H, relu=False)

    sums1, hist = sc_agg1(x, edge_index, srcp, dstp)
    cnt = jnp.sum(hist, axis=0).reshape(-1, 1)
    h1 = tc1(sums1, cnt, x, W_l1, b_l1[None, :], W_r1)
    (sums2,) = sc_agg2(h1, edge_index, srcp, dstp)
    out = tc2(sums2, cnt, h1, W_l2, b_l2[None, :], W_r2)
    return out
